# baseline (speedup 1.0000x reference)
import functools

import jax
import jax.numpy as jnp
from jax import lax
from jax.experimental import pallas as pl
from jax.experimental.pallas import tpu as pltpu
from jax.experimental.pallas import tpu_sc as plsc

_BLK = 4096
_QH = 1 << 18


def _bf16_bits(a):
    u = jax.lax.bitcast_convert_type(a, jnp.uint32)
    return (u + 0x7FFF + ((u >> 16) & 1)) >> 16


def _tc_premul(table_t, W_lo, W_hi):
    C, R = table_t.shape
    nb = _QH // _BLK
    last_blk = (R - 1) // _BLK

    def body(t0_ref, t1_ref, t2_ref, t3_ref, wlo_ref, whi_ref, o_ref):
        tcat = jnp.concatenate(
            [t0_ref[...], t1_ref[...], t2_ref[...], t3_ref[...]], axis=0
        ).astype(jnp.bfloat16)
        dn = (((0,), (0,)), ((), ()))
        lo = jax.lax.dot_general(
            tcat, wlo_ref[...], dn, preferred_element_type=jnp.float32
        )
        hi = jax.lax.dot_general(
            tcat, whi_ref[...], dn, preferred_element_type=jnp.float32
        )
        o_ref[...] = _bf16_bits(lo) | (_bf16_bits(hi) << 16)

    def in_map(q):
        return lambda i: (0, jnp.minimum(i + q * nb, last_blk))

    return pl.pallas_call(
        body,
        grid=(nb,),
        in_specs=[
            pl.BlockSpec((C, _BLK), in_map(0)),
            pl.BlockSpec((C, _BLK), in_map(1)),
            pl.BlockSpec((C, _BLK), in_map(2)),
            pl.BlockSpec((C, _BLK), in_map(3)),
            pl.BlockSpec((4 * C, 2 * C), lambda i: (0, 0)),
            pl.BlockSpec((4 * C, 2 * C), lambda i: (0, 0)),
        ],
        out_specs=pl.BlockSpec((_BLK, 128), lambda i: (i, 0)),
        out_shape=jax.ShapeDtypeStruct((_QH, 128), jnp.uint32),
        compiler_params=pltpu.CompilerParams(vmem_limit_bytes=120 * 1024 * 1024),
    )(table_t, table_t, table_t, table_t, W_lo, W_hi)


def _sc_gather(idx, mm_flat):
    B, = idx.shape
    N, D = mm_flat.shape
    info = plsc.get_sparse_core_info()
    NC, NS = info.num_cores, info.num_subcores
    NW = NC * NS
    b_per_w = B // NW

    mesh = plsc.VectorSubcoreMesh(core_axis_name="c", subcore_axis_name="s")

    @functools.partial(
        pl.kernel,
        mesh=mesh,
        out_type=jax.ShapeDtypeStruct((B, D), jnp.uint32),
        scratch_types=[
            pltpu.VMEM((b_per_w,), jnp.int32),
            pltpu.VMEM((b_per_w, D), jnp.uint32),
            pltpu.SemaphoreType.DMA,
        ],
        compiler_params=pltpu.CompilerParams(use_tc_tiling_on_sc=False),
    )
    def k(idx_hbm, mm_hbm, out_hbm, idx_v, rows_v, sem):
        wid = lax.axis_index("s") * NC + lax.axis_index("c")
        base = wid * b_per_w
        pltpu.sync_copy(idx_hbm.at[pl.ds(base, b_per_w)], idx_v)
        pltpu.async_copy(mm_hbm.at[idx_v], rows_v, sem).wait()
        pltpu.sync_copy(rows_v, out_hbm.at[pl.ds(base, b_per_w)])

    return k(idx, mm_flat)


def _tc_bias_tanh(zp, b_lo, b_hi):
    N, D2 = zp.shape
    BLK = 1024

    def body(z_ref, blo_ref, bhi_ref, o_ref):
        z = z_ref[...]
        lo = jax.lax.bitcast_convert_type(z << 16, jnp.float32)
        hi = jax.lax.bitcast_convert_type(z & jnp.uint32(0xFFFF0000), jnp.float32)
        ylo = jnp.tanh(lo + blo_ref[...])
        yhi = jnp.tanh(hi + bhi_ref[...])
        parts = []
        for q in range(4):
            parts.append(ylo[:, 32 * q : 32 * (q + 1)])
            parts.append(yhi[:, 32 * q : 32 * (q + 1)])
        o_ref[...] = jnp.concatenate(parts, axis=1)

    return pl.pallas_call(
        body,
        grid=(N // BLK,),
        in_specs=[
            pl.BlockSpec((BLK, D2), lambda i: (i, 0)),
            pl.BlockSpec((1, D2), lambda i: (0, 0)),
            pl.BlockSpec((1, D2), lambda i: (0, 0)),
        ],
        out_specs=pl.BlockSpec((BLK, 2 * D2), lambda i: (i, 0)),
        out_shape=jax.ShapeDtypeStruct((N, 2 * D2), jnp.float32),
    )(zp, b_lo, b_hi)


def kernel(x, table, W, b):
    B = x.shape[0]
    R, D = table.shape
    eye4 = jnp.eye(4, dtype=W.dtype)
    W_lo = jnp.kron(eye4, W[: D // 2].T).astype(jnp.bfloat16)
    W_hi = jnp.kron(eye4, W[D // 2 :].T).astype(jnp.bfloat16)
    mm = _tc_premul(table.T, W_lo, W_hi)
    last = x[:, -1].astype(jnp.int32)
    fidx = 4 * (last & (_QH - 1)) + (last >> 18)
    mm_flat = mm.reshape(4 * _QH, D // 2)
    z = _sc_gather(fidx, mm_flat)
    zp = z.reshape(B // 4, 128)
    b_lo = jnp.tile(b[:32], 4).reshape(1, 128)
    b_hi = jnp.tile(b[32:], 4).reshape(1, 128)
    out = _tc_bias_tanh(zp, b_lo, b_hi)
    return out.reshape(B, D)


# speedup vs baseline: 1.0921x; 1.0921x over previous
import functools

import jax
import jax.numpy as jnp
from jax import lax
from jax.experimental import pallas as pl
from jax.experimental.pallas import tpu as pltpu
from jax.experimental.pallas import tpu_sc as plsc

_BLK = 16384
_QH = 1 << 18


def _bf16_bits(a):
    u = jax.lax.bitcast_convert_type(a, jnp.uint32)
    return (u + 0x7FFF + ((u >> 16) & 1)) >> 16


def _tc_premul(table_t, W_lo, W_hi):
    C, R = table_t.shape
    nb = _QH // _BLK
    last_blk = (R - 1) // _BLK

    def body(t0_ref, t1_ref, t2_ref, t3_ref, wlo_ref, whi_ref, o_ref):
        tcat = jnp.concatenate(
            [t0_ref[...], t1_ref[...], t2_ref[...], t3_ref[...]], axis=0
        ).astype(jnp.bfloat16)
        dn = (((0,), (0,)), ((), ()))
        lo = jax.lax.dot_general(
            tcat, wlo_ref[...], dn, preferred_element_type=jnp.float32
        )
        hi = jax.lax.dot_general(
            tcat, whi_ref[...], dn, preferred_element_type=jnp.float32
        )
        o_ref[...] = _bf16_bits(lo) | (_bf16_bits(hi) << 16)

    def in_map(q):
        return lambda i: (0, jnp.minimum(i + q * nb, last_blk))

    return pl.pallas_call(
        body,
        grid=(nb,),
        in_specs=[
            pl.BlockSpec((C, _BLK), in_map(0)),
            pl.BlockSpec((C, _BLK), in_map(1)),
            pl.BlockSpec((C, _BLK), in_map(2)),
            pl.BlockSpec((C, _BLK), in_map(3)),
            pl.BlockSpec((4 * C, 2 * C), lambda i: (0, 0)),
            pl.BlockSpec((4 * C, 2 * C), lambda i: (0, 0)),
        ],
        out_specs=pl.BlockSpec((_BLK, 128), lambda i: (i, 0)),
        out_shape=jax.ShapeDtypeStruct((_QH, 128), jnp.uint32),
        compiler_params=pltpu.CompilerParams(vmem_limit_bytes=120 * 1024 * 1024),
    )(table_t, table_t, table_t, table_t, W_lo, W_hi)


def _sc_gather(idx, mm_flat):
    B, = idx.shape
    N, D = mm_flat.shape
    info = plsc.get_sparse_core_info()
    NC, NS = info.num_cores, info.num_subcores
    NW = NC * NS
    b_per_w = B // NW

    mesh = plsc.VectorSubcoreMesh(core_axis_name="c", subcore_axis_name="s")

    @functools.partial(
        pl.kernel,
        mesh=mesh,
        out_type=jax.ShapeDtypeStruct((B, D), jnp.uint32),
        scratch_types=[
            pltpu.VMEM((b_per_w,), jnp.int32),
            pltpu.VMEM((b_per_w, D), jnp.uint32),
            pltpu.SemaphoreType.DMA,
        ],
        compiler_params=pltpu.CompilerParams(use_tc_tiling_on_sc=False),
    )
    def k(idx_hbm, mm_hbm, out_hbm, idx_v, rows_v, sem):
        wid = lax.axis_index("s") * NC + lax.axis_index("c")
        base = wid * b_per_w
        pltpu.sync_copy(idx_hbm.at[pl.ds(base, b_per_w)], idx_v)
        pltpu.async_copy(mm_hbm.at[idx_v], rows_v, sem).wait()
        pltpu.sync_copy(rows_v, out_hbm.at[pl.ds(base, b_per_w)])

    return k(idx, mm_flat)


def _tc_bias_tanh(zp, b_lo, b_hi):
    N, D2 = zp.shape
    BLK = 1024

    def body(z_ref, blo_ref, bhi_ref, o_ref):
        z = z_ref[...]
        lo = jax.lax.bitcast_convert_type(z << 16, jnp.float32)
        hi = jax.lax.bitcast_convert_type(z & jnp.uint32(0xFFFF0000), jnp.float32)
        ylo = jnp.tanh(lo + blo_ref[...])
        yhi = jnp.tanh(hi + bhi_ref[...])
        parts = []
        for q in range(4):
            parts.append(ylo[:, 32 * q : 32 * (q + 1)])
            parts.append(yhi[:, 32 * q : 32 * (q + 1)])
        o_ref[...] = jnp.concatenate(parts, axis=1)

    return pl.pallas_call(
        body,
        grid=(N // BLK,),
        in_specs=[
            pl.BlockSpec((BLK, D2), lambda i: (i, 0)),
            pl.BlockSpec((1, D2), lambda i: (0, 0)),
            pl.BlockSpec((1, D2), lambda i: (0, 0)),
        ],
        out_specs=pl.BlockSpec((BLK, 2 * D2), lambda i: (i, 0)),
        out_shape=jax.ShapeDtypeStruct((N, 2 * D2), jnp.float32),
    )(zp, b_lo, b_hi)


def kernel(x, table, W, b):
    B = x.shape[0]
    R, D = table.shape
    eye4 = jnp.eye(4, dtype=W.dtype)
    W_lo = jnp.kron(eye4, W[: D // 2].T).astype(jnp.bfloat16)
    W_hi = jnp.kron(eye4, W[D // 2 :].T).astype(jnp.bfloat16)
    mm = _tc_premul(table.T, W_lo, W_hi)
    last = x[:, -1].astype(jnp.int32)
    fidx = 4 * (last & (_QH - 1)) + (last >> 18)
    mm_flat = mm.reshape(4 * _QH, D // 2)
    z = _sc_gather(fidx, mm_flat)
    zp = z.reshape(B // 4, 128)
    b_lo = jnp.tile(b[:32], 4).reshape(1, 128)
    b_hi = jnp.tile(b[32:], 4).reshape(1, 128)
    out = _tc_bias_tanh(zp, b_lo, b_hi)
    return out.reshape(B, D)
